# baseline (speedup 1.0000x reference)
import functools

import jax
import jax.numpy as jnp
from jax import lax
from jax.experimental import pallas as pl
from jax.experimental.pallas import tpu as pltpu
from jax.experimental.pallas import tpu_sc as plsc

BB, FIN, FOUT, NN, KK = 4, 128, 256, 2048, 16
NBA = 256
NB = 256
EDGES = BB * NN * KK
M_EDGES = float(EDGES)
EPS = 1e-5
INF = 1e30
BIGI = 2 ** 30

NW = 32
CHUNK = 128
NCH_W = EDGES // (NW * CHUNK)


def _lrelu(x):
    return jnp.where(x >= 0, x, 0.2 * x)


def _dot(a, b):
    return lax.dot_general(a, b, (((1,), (0,)), ((), ())),
                           precision=lax.Precision.DEFAULT,
                           preferred_element_type=jnp.float32)


def _prep_body(xt_ref, x_ref, st_ref, w1t_ref, wxt_ref, wst_ref, bx_ref,
               bs_ref, idx_ref, y1_ref, a_ref, s_ref):
    xtb = xt_ref[0]
    xb = x_ref[0]
    prod = lax.dot_general(xtb, xb, (((1,), (0,)), ((), ())),
                           precision=lax.Precision.DEFAULT,
                           preferred_element_type=jnp.float32)
    cn = jnp.sum(xb * xb, axis=0, keepdims=True)
    rn = jnp.sum(xtb * xtb, axis=1, keepdims=True)
    d = -2.0 * prod + rn + cn
    i = pl.program_id(1)
    col = lax.broadcasted_iota(jnp.int32, (NBA, NN), 1)
    row = lax.broadcasted_iota(jnp.int32, (NBA, NN), 0) + i * NBA
    d = jnp.where(col == row, INF, d)
    colf = col.astype(jnp.float32)
    for k in range(KK):
        m = jnp.min(d, axis=1, keepdims=True)
        cand = jnp.where(d == m, colf, 1e9)
        jm = jnp.min(cand, axis=1, keepdims=True)
        idx_ref[0, :, k:k + 1] = jm.astype(jnp.int32)
        d = jnp.where(colf == jm, INF, d)
    y1_ref[0] = _dot(xtb, w1t_ref[...])
    wxt = wxt_ref[...]
    a_ref[0] = _dot(xtb, wxt[:FIN] - wxt[FIN:]) + bx_ref[...]
    s_ref[0] = _dot(st_ref[0], wst_ref[...]) + bs_ref[...]


def _prep_call(xt, x, stt, w1t, wxt, wst, bxr, bsr):
    nblk = NN // NBA
    bbl = xt.shape[0]
    return pl.pallas_call(
        _prep_body,
        grid=(bbl, nblk),
        in_specs=[
            pl.BlockSpec((1, NBA, FIN), lambda b, i: (b, i, 0)),
            pl.BlockSpec((1, FIN, NN), lambda b, i: (b, 0, 0)),
            pl.BlockSpec((1, NBA, FOUT), lambda b, i: (b, i, 0)),
            pl.BlockSpec((FIN, FIN), lambda b, i: (0, 0)),
            pl.BlockSpec((2 * FIN, FOUT), lambda b, i: (0, 0)),
            pl.BlockSpec((FOUT, 2 * FOUT), lambda b, i: (0, 0)),
            pl.BlockSpec((1, FOUT), lambda b, i: (0, 0)),
            pl.BlockSpec((1, 2 * FOUT), lambda b, i: (0, 0)),
        ],
        out_specs=[
            pl.BlockSpec((1, NBA, KK), lambda b, i: (b, i, 0)),
            pl.BlockSpec((1, NBA, FIN), lambda b, i: (b, i, 0)),
            pl.BlockSpec((1, NBA, FOUT), lambda b, i: (b, i, 0)),
            pl.BlockSpec((1, NBA, 2 * FOUT), lambda b, i: (b, i, 0)),
        ],
        out_shape=[
            jax.ShapeDtypeStruct((bbl, NN, KK), jnp.int32),
            jax.ShapeDtypeStruct((bbl, NN, FIN), jnp.float32),
            jax.ShapeDtypeStruct((bbl, NN, FOUT), jnp.float32),
            jax.ShapeDtypeStruct((bbl, NN, 2 * FOUT), jnp.float32),
        ],
    )(xt, x, stt, w1t, wxt, wst, bxr, bsr)


def _gather_call(table, idxw):
    mesh = plsc.VectorSubcoreMesh(core_axis_name="c", subcore_axis_name="s")
    nch = idxw.shape[1]

    @functools.partial(
        pl.kernel,
        out_type=jax.ShapeDtypeStruct((NW * nch * CHUNK, FIN), jnp.float32),
        mesh=mesh,
        scratch_types=[
            pltpu.VMEM((nch, CHUNK), jnp.int32),
            pltpu.VMEM((CHUNK, FIN), jnp.float32),
            pltpu.VMEM((CHUNK, FIN), jnp.float32),
            pltpu.SemaphoreType.DMA,
            pltpu.SemaphoreType.DMA,
        ],
    )
    def gk(table_hbm, idx_hbm, out_hbm, idx_v, rows_a, rows_b, sem_a, sem_b):
        wid = lax.axis_index("s") * 2 + lax.axis_index("c")
        pltpu.sync_copy(idx_hbm.at[wid], idx_v)
        base = wid * (nch * CHUNK)
        pltpu.async_copy(table_hbm.at[idx_v.at[0]], rows_a, sem_a)
        pltpu.async_copy(table_hbm.at[idx_v.at[1]], rows_b, sem_b)

        def body(i, carry):
            c = 2 * i
            pltpu.make_async_copy(table_hbm.at[idx_v.at[c]], rows_a,
                                  sem_a).wait()
            pltpu.sync_copy(rows_a, out_hbm.at[pl.ds(base + c * CHUNK, CHUNK)])

            @pl.when(c + 2 < nch)
            def _():
                pltpu.async_copy(table_hbm.at[idx_v.at[c + 2]], rows_a, sem_a)

            pltpu.make_async_copy(table_hbm.at[idx_v.at[c + 1]], rows_b,
                                  sem_b).wait()
            pltpu.sync_copy(rows_b,
                            out_hbm.at[pl.ds(base + (c + 1) * CHUNK, CHUNK)])

            @pl.when(c + 3 < nch)
            def _():
                pltpu.async_copy(table_hbm.at[idx_v.at[c + 3]], rows_b, sem_b)

            return carry

        lax.fori_loop(0, nch // 2, body, 0)

    return gk(table, idxw)


def _edge_blocks(xg_ref, w1t, b1, wdt):
    xgf = xg_ref[0].reshape(KK * NB, FIN)
    hf = (_dot(xgf, w1t) + b1).reshape(KK, NB, FIN)
    xf = _dot(xgf, wdt).reshape(KK, NB, FOUT)
    return hf, xf


def _stats1_body(xg_ref, y1_ref, a_ref, w1t_ref, wxt_ref, b1_ref,
                 sh_ref, sx_ref):
    y1b = y1_ref[0]
    ab = a_ref[0]
    hf, xf = _edge_blocks(xg_ref, w1t_ref[...], b1_ref[...],
                          wxt_ref[...][FIN:])
    sh = jnp.zeros((1, FIN), jnp.float32)
    sh2 = jnp.zeros((1, FIN), jnp.float32)
    sx = jnp.zeros((1, FOUT), jnp.float32)
    sx2 = jnp.zeros((1, FOUT), jnp.float32)
    for k in range(KK):
        h = hf[k] - y1b
        sh = sh + jnp.sum(h, axis=0, keepdims=True)
        sh2 = sh2 + jnp.sum(h * h, axis=0, keepdims=True)
        xx = xf[k] + ab
        sx = sx + jnp.sum(xx, axis=0, keepdims=True)
        sx2 = sx2 + jnp.sum(xx * xx, axis=0, keepdims=True)
    ph = jnp.concatenate([sh, sh2, jnp.zeros((6, FIN), jnp.float32)], axis=0)
    px = jnp.concatenate([sx, sx2, jnp.zeros((6, FOUT), jnp.float32)], axis=0)
    first = (pl.program_id(0) == 0) & (pl.program_id(1) == 0)

    @pl.when(first)
    def _():
        sh_ref[...] = jnp.zeros_like(sh_ref)
        sx_ref[...] = jnp.zeros_like(sx_ref)

    sh_ref[...] += ph
    sx_ref[...] += px


def _stats1_call(xg4, y1t, at, w1t, wxt, b1r):
    nblk = NN // NB
    bbl = xg4.shape[0]
    return pl.pallas_call(
        _stats1_body,
        grid=(bbl, nblk),
        in_specs=[
            pl.BlockSpec((1, KK, NB, FIN), lambda b, i: (b, 0, i, 0)),
            pl.BlockSpec((1, NB, FIN), lambda b, i: (b, i, 0)),
            pl.BlockSpec((1, NB, FOUT), lambda b, i: (b, i, 0)),
            pl.BlockSpec((FIN, FIN), lambda b, i: (0, 0)),
            pl.BlockSpec((2 * FIN, FOUT), lambda b, i: (0, 0)),
            pl.BlockSpec((1, FIN), lambda b, i: (0, 0)),
        ],
        out_specs=[
            pl.BlockSpec((8, FIN), lambda b, i: (0, 0)),
            pl.BlockSpec((8, FOUT), lambda b, i: (0, 0)),
        ],
        out_shape=[
            jax.ShapeDtypeStruct((8, FIN), jnp.float32),
            jax.ShapeDtypeStruct((8, FOUT), jnp.float32),
        ],
    )(xg4, y1t, at, w1t, wxt, b1r)


def _bn_consts(sums, g, be, count):
    m = sums[0:1, :] / count
    v = sums[1:2, :] / count - m * m
    a = g / jnp.sqrt(v + EPS)
    return m, a, be


def _stats2_body(xg_ref, y1_ref, w1t_ref, b1_ref, shs_ref, g1_ref, be1_ref,
                 w2t_ref, b2_ref, shw_ref):
    y1b = y1_ref[0]
    m1, a1, be1 = _bn_consts(shs_ref[...], g1_ref[...], be1_ref[...], M_EDGES)
    xgf = xg_ref[0].reshape(KK * NB, FIN)
    hf = (_dot(xgf, w1t_ref[...]) + b1_ref[...]).reshape(KK, NB, FIN)
    u_list = [_lrelu(((hf[k] - y1b) - m1) * a1 + be1) for k in range(KK)]
    uf = jnp.concatenate(u_list, axis=0)
    hwf = _dot(uf, w2t_ref[...]) + b2_ref[...]
    s = jnp.sum(hwf, axis=0, keepdims=True)
    s2 = jnp.sum(hwf * hwf, axis=0, keepdims=True)
    p = jnp.concatenate([s, s2, jnp.zeros((6, FOUT), jnp.float32)], axis=0)
    first = (pl.program_id(0) == 0) & (pl.program_id(1) == 0)

    @pl.when(first)
    def _():
        shw_ref[...] = jnp.zeros_like(shw_ref)

    shw_ref[...] += p


def _stats2_call(xg4, y1t, w1t, b1r, shs, g1r, be1r, w2t, b2r):
    nblk = NN // NB
    bbl = xg4.shape[0]
    return pl.pallas_call(
        _stats2_body,
        grid=(bbl, nblk),
        in_specs=[
            pl.BlockSpec((1, KK, NB, FIN), lambda b, i: (b, 0, i, 0)),
            pl.BlockSpec((1, NB, FIN), lambda b, i: (b, i, 0)),
            pl.BlockSpec((FIN, FIN), lambda b, i: (0, 0)),
            pl.BlockSpec((1, FIN), lambda b, i: (0, 0)),
            pl.BlockSpec((8, FIN), lambda b, i: (0, 0)),
            pl.BlockSpec((1, FIN), lambda b, i: (0, 0)),
            pl.BlockSpec((1, FIN), lambda b, i: (0, 0)),
            pl.BlockSpec((FIN, FOUT), lambda b, i: (0, 0)),
            pl.BlockSpec((1, FOUT), lambda b, i: (0, 0)),
        ],
        out_specs=[pl.BlockSpec((8, FOUT), lambda b, i: (0, 0))],
        out_shape=[jax.ShapeDtypeStruct((8, FOUT), jnp.float32)],
    )(xg4, y1t, w1t, b1r, shs, g1r, be1r, w2t, b2r)[0]


def _main_body(xg_ref, y1_ref, a_ref, w1t_ref, wxt_ref, b1_ref,
               shs_ref, g1_ref, be1_ref, w2t_ref, b2_ref,
               shw_ref, g2_ref, be2_ref, sxs_ref, gx_ref, bex_ref,
               wot_ref, bo_ref, o_ref, ast_ref):
    y1b = y1_ref[0]
    ab = a_ref[0]
    m1, a1, be1 = _bn_consts(shs_ref[...], g1_ref[...], be1_ref[...], M_EDGES)
    m2, a2, be2 = _bn_consts(shw_ref[...], g2_ref[...], be2_ref[...], M_EDGES)
    m3, a3, bex = _bn_consts(sxs_ref[...], gx_ref[...], bex_ref[...], M_EDGES)
    hf, xf = _edge_blocks(xg_ref, w1t_ref[...], b1_ref[...],
                          wxt_ref[...][FIN:])
    u_list = [_lrelu(((hf[k] - y1b) - m1) * a1 + be1) for k in range(KK)]
    uf = jnp.concatenate(u_list, axis=0)
    hwf = _dot(uf, w2t_ref[...]) + b2_ref[...]
    hw4 = hwf.reshape(KK, NB, FOUT)
    z_list = []
    xxn_list = []
    for k in range(KK):
        z_list.append(_lrelu((hw4[k] - m2) * a2 + be2))
        xx = xf[k] + ab
        xxn_list.append(_lrelu((xx - m3) * a3 + bex))
    mx = z_list[0]
    for k in range(1, KK):
        mx = jnp.maximum(mx, z_list[k])
    e_list = []
    s = jnp.zeros((NB, FOUT), jnp.float32)
    for k in range(KK):
        e = jnp.exp(z_list[k] - mx)
        e_list.append(e)
        s = s + e
    rs = 1.0 / s
    acc = jnp.zeros((NB, FOUT), jnp.float32)
    for k in range(KK):
        acc = acc + _dot(xxn_list[k] * e_list[k] * rs, wot_ref[k])
    out = acc + bo_ref[...]
    o_ref[0] = out
    so = jnp.sum(out, axis=0, keepdims=True)
    so2 = jnp.sum(out * out, axis=0, keepdims=True)
    p = jnp.concatenate([so, so2, jnp.zeros((6, FOUT), jnp.float32)], axis=0)
    first = pl.program_id(1) == 0

    @pl.when(first)
    def _():
        ast_ref[...] = jnp.zeros_like(ast_ref)

    ast_ref[...] += p[None]


def _main_call(xg4, y1t, at, w1t, wxt, b1r, shs, g1r, be1r, w2t, b2r,
               shw, g2r, be2r, sxs, gxr, bexr, wot, bor):
    nblk = NN // NB
    bbl = xg4.shape[0]
    cfull = lambda b, i: (0, 0)
    return pl.pallas_call(
        _main_body,
        grid=(bbl, nblk),
        in_specs=[
            pl.BlockSpec((1, KK, NB, FIN), lambda b, i: (b, 0, i, 0)),
            pl.BlockSpec((1, NB, FIN), lambda b, i: (b, i, 0)),
            pl.BlockSpec((1, NB, FOUT), lambda b, i: (b, i, 0)),
            pl.BlockSpec((FIN, FIN), cfull),
            pl.BlockSpec((2 * FIN, FOUT), cfull),
            pl.BlockSpec((1, FIN), cfull),
            pl.BlockSpec((8, FIN), cfull),
            pl.BlockSpec((1, FIN), cfull),
            pl.BlockSpec((1, FIN), cfull),
            pl.BlockSpec((FIN, FOUT), cfull),
            pl.BlockSpec((1, FOUT), cfull),
            pl.BlockSpec((8, FOUT), cfull),
            pl.BlockSpec((1, FOUT), cfull),
            pl.BlockSpec((1, FOUT), cfull),
            pl.BlockSpec((8, FOUT), cfull),
            pl.BlockSpec((1, FOUT), cfull),
            pl.BlockSpec((1, FOUT), cfull),
            pl.BlockSpec((KK, FOUT, FOUT), lambda b, i: (0, 0, 0)),
            pl.BlockSpec((1, FOUT), cfull),
        ],
        out_specs=[
            pl.BlockSpec((1, NB, FOUT), lambda b, i: (b, i, 0)),
            pl.BlockSpec((1, 8, FOUT), lambda b, i: (b, 0, 0)),
        ],
        out_shape=[
            jax.ShapeDtypeStruct((bbl, NN, FOUT), jnp.float32),
            jax.ShapeDtypeStruct((bbl, 8, FOUT), jnp.float32),
        ],
    )(xg4, y1t, at, w1t, wxt, b1r, shs, g1r, be1r, w2t, b2r,
      shw, g2r, be2r, sxs, gxr, bexr, wot, bor)


def _ada_body(o_ref, st_ref, ast_ref, out_ref):
    stats = ast_ref[0]
    m = stats[0:1, :] / float(NN)
    v = stats[1:2, :] / float(NN) - m * m
    o = o_ref[0]
    sv = st_ref[0]
    gamma = sv[:, :FOUT]
    beta = sv[:, FOUT:]
    y = gamma * (o - m) / jnp.sqrt(v + EPS) + beta
    out_ref[0] = _lrelu(y)


def _ada_call(o, st, ast):
    nblk = NN // NB
    bbl = o.shape[0]
    return pl.pallas_call(
        _ada_body,
        grid=(bbl, nblk),
        in_specs=[
            pl.BlockSpec((1, NB, FOUT), lambda b, i: (b, i, 0)),
            pl.BlockSpec((1, NB, 2 * FOUT), lambda b, i: (b, i, 0)),
            pl.BlockSpec((1, 8, FOUT), lambda b, i: (b, 0, 0)),
        ],
        out_specs=pl.BlockSpec((1, NB, FOUT), lambda b, i: (b, i, 0)),
        out_shape=jax.ShapeDtypeStruct((bbl, NN, FOUT), jnp.float32),
    )(o, st, ast)


def kernel(x, style, W1, b1, g1, be1, W2, b2, g2, be2, Wx, bx, gx, bex,
           Wo, bo, Ws, bs):
    xt = jnp.transpose(x, (0, 2, 1))
    stt = jnp.transpose(style, (0, 2, 1))
    w1t = W1.T
    wxt = Wx.T
    w2t = W2.T
    wst = Ws.T
    wot = jnp.transpose(Wo, (2, 1, 0))
    b1r = b1.reshape(1, FIN)
    g1r = g1.reshape(1, FIN)
    be1r = be1.reshape(1, FIN)
    b2r = b2.reshape(1, FOUT)
    g2r = g2.reshape(1, FOUT)
    be2r = be2.reshape(1, FOUT)
    bxr = bx.reshape(1, FOUT)
    gxr = gx.reshape(1, FOUT)
    bexr = bex.reshape(1, FOUT)
    bor = bo.reshape(1, FOUT)
    bsr = bs.reshape(1, 2 * FOUT)

    table = xt.reshape(BB * NN, FIN)
    hb = BB // 2
    nch_h = NCH_W // 2
    prep = []
    xg4s = []
    for hsel in range(2):
        sl = slice(hb * hsel, hb * (hsel + 1))
        idx_h, y1t_h, at_h, st_h = _prep_call(xt[sl], x[sl], stt[sl],
                                              w1t, wxt, wst, bxr, bsr)
        prep.append((y1t_h, at_h, st_h))
        off = ((jnp.arange(hb, dtype=jnp.int32) + hb * hsel) * NN
               ).reshape(hb, 1, 1)
        flat_h = (jnp.transpose(idx_h, (0, 2, 1)) + off
                  ).reshape(NW, nch_h, CHUNK)
        xg_h = _gather_call(table, flat_h)
        xg4s.append(xg_h.reshape(hb, KK, NN, FIN))

    parts1 = [_stats1_call(xg4s[hsel], prep[hsel][0], prep[hsel][1],
                           w1t, wxt, b1r) for hsel in range(2)]
    shs = parts1[0][0] + parts1[1][0]
    sxs = parts1[0][1] + parts1[1][1]
    shw = sum(_stats2_call(xg4s[hsel], prep[hsel][0], w1t, b1r, shs,
                           g1r, be1r, w2t, b2r) for hsel in range(2))
    outs = []
    for hsel in range(2):
        o_h, ast_h = _main_call(xg4s[hsel], prep[hsel][0], prep[hsel][1],
                                w1t, wxt, b1r, shs, g1r, be1r, w2t, b2r,
                                shw, g2r, be2r, sxs, gxr, bexr, wot, bor)
        outs.append(_ada_call(o_h, prep[hsel][2], ast_h))
    outf = jnp.concatenate(outs, axis=0)
    return jnp.transpose(outf, (0, 2, 1))


# speedup vs baseline: 1.0026x; 1.0026x over previous
import functools

import jax
import jax.numpy as jnp
from jax import lax
from jax.experimental import pallas as pl
from jax.experimental.pallas import tpu as pltpu
from jax.experimental.pallas import tpu_sc as plsc

BB, FIN, FOUT, NN, KK = 4, 128, 256, 2048, 16
NBA = 256
NB = 256
EDGES = BB * NN * KK
M_EDGES = float(EDGES)
EPS = 1e-5
INF = 1e30
BIGI = 2 ** 30

NW = 32
CHUNK = 128
NCH_W = EDGES // (NW * CHUNK)


def _lrelu(x):
    return jnp.where(x >= 0, x, 0.2 * x)


def _dot(a, b):
    return lax.dot_general(a, b, (((1,), (0,)), ((), ())),
                           precision=lax.Precision.DEFAULT,
                           preferred_element_type=jnp.float32)


def _prep_body(xt_ref, x_ref, st_ref, w1t_ref, wxt_ref, wst_ref, bx_ref,
               bs_ref, idx_ref, y1_ref, a_ref, s_ref):
    xtb = xt_ref[0]
    xb = x_ref[0]
    prod = lax.dot_general(xtb, xb, (((1,), (0,)), ((), ())),
                           precision=lax.Precision.DEFAULT,
                           preferred_element_type=jnp.float32)
    cn = jnp.sum(xb * xb, axis=0, keepdims=True)
    rn = jnp.sum(xtb * xtb, axis=1, keepdims=True)
    d = -2.0 * prod + rn + cn
    i = pl.program_id(1)
    col = lax.broadcasted_iota(jnp.int32, (NBA, NN), 1)
    row = lax.broadcasted_iota(jnp.int32, (NBA, NN), 0) + i * NBA
    d = jnp.where(col == row, INF, d)
    colf = col.astype(jnp.float32)
    for k in range(KK):
        m = jnp.min(d, axis=1, keepdims=True)
        cand = jnp.where(d == m, colf, 1e9)
        jm = jnp.min(cand, axis=1, keepdims=True)
        idx_ref[0, :, k:k + 1] = jm.astype(jnp.int32)
        d = jnp.where(colf == jm, INF, d)
    y1_ref[0] = _dot(xtb, w1t_ref[...])
    wxt = wxt_ref[...]
    a_ref[0] = _dot(xtb, wxt[:FIN] - wxt[FIN:]) + bx_ref[...]
    s_ref[0] = _dot(st_ref[0], wst_ref[...]) + bs_ref[...]


def _prep_call(xt, x, stt, w1t, wxt, wst, bxr, bsr):
    nblk = NN // NBA
    return pl.pallas_call(
        _prep_body,
        grid=(BB, nblk),
        in_specs=[
            pl.BlockSpec((1, NBA, FIN), lambda b, i: (b, i, 0)),
            pl.BlockSpec((1, FIN, NN), lambda b, i: (b, 0, 0)),
            pl.BlockSpec((1, NBA, FOUT), lambda b, i: (b, i, 0)),
            pl.BlockSpec((FIN, FIN), lambda b, i: (0, 0)),
            pl.BlockSpec((2 * FIN, FOUT), lambda b, i: (0, 0)),
            pl.BlockSpec((FOUT, 2 * FOUT), lambda b, i: (0, 0)),
            pl.BlockSpec((1, FOUT), lambda b, i: (0, 0)),
            pl.BlockSpec((1, 2 * FOUT), lambda b, i: (0, 0)),
        ],
        out_specs=[
            pl.BlockSpec((1, NBA, KK), lambda b, i: (b, i, 0)),
            pl.BlockSpec((1, NBA, FIN), lambda b, i: (b, i, 0)),
            pl.BlockSpec((1, NBA, FOUT), lambda b, i: (b, i, 0)),
            pl.BlockSpec((1, NBA, 2 * FOUT), lambda b, i: (b, i, 0)),
        ],
        out_shape=[
            jax.ShapeDtypeStruct((BB, NN, KK), jnp.int32),
            jax.ShapeDtypeStruct((BB, NN, FIN), jnp.float32),
            jax.ShapeDtypeStruct((BB, NN, FOUT), jnp.float32),
            jax.ShapeDtypeStruct((BB, NN, 2 * FOUT), jnp.float32),
        ],
    )(xt, x, stt, w1t, wxt, wst, bxr, bsr)


def _gather_call(table, idxw):
    mesh = plsc.VectorSubcoreMesh(core_axis_name="c", subcore_axis_name="s")

    @functools.partial(
        pl.kernel,
        out_type=jax.ShapeDtypeStruct((EDGES, FIN), jnp.float32),
        mesh=mesh,
        scratch_types=[
            pltpu.VMEM((NCH_W, CHUNK), jnp.int32),
            pltpu.VMEM((CHUNK, FIN), jnp.float32),
            pltpu.VMEM((CHUNK, FIN), jnp.float32),
            pltpu.SemaphoreType.DMA,
            pltpu.SemaphoreType.DMA,
        ],
    )
    def gk(table_hbm, idx_hbm, out_hbm, idx_v, rows_a, rows_b, sem_a, sem_b):
        wid = lax.axis_index("s") * 2 + lax.axis_index("c")
        pltpu.sync_copy(idx_hbm.at[wid], idx_v)
        base = wid * (NCH_W * CHUNK)
        pltpu.async_copy(table_hbm.at[idx_v.at[0]], rows_a, sem_a)
        pltpu.async_copy(table_hbm.at[idx_v.at[1]], rows_b, sem_b)

        def body(i, carry):
            c = 2 * i
            pltpu.make_async_copy(table_hbm.at[idx_v.at[c]], rows_a,
                                  sem_a).wait()
            pltpu.sync_copy(rows_a, out_hbm.at[pl.ds(base + c * CHUNK, CHUNK)])

            @pl.when(c + 2 < NCH_W)
            def _():
                pltpu.async_copy(table_hbm.at[idx_v.at[c + 2]], rows_a, sem_a)

            pltpu.make_async_copy(table_hbm.at[idx_v.at[c + 1]], rows_b,
                                  sem_b).wait()
            pltpu.sync_copy(rows_b,
                            out_hbm.at[pl.ds(base + (c + 1) * CHUNK, CHUNK)])

            @pl.when(c + 3 < NCH_W)
            def _():
                pltpu.async_copy(table_hbm.at[idx_v.at[c + 3]], rows_b, sem_b)

            return carry

        lax.fori_loop(0, NCH_W // 2, body, 0)

    return gk(table, idxw)


def _edge_blocks(xg_ref, w1t, b1, wdt):
    xgf = xg_ref[0].reshape(KK * NB, FIN)
    hf = (_dot(xgf, w1t) + b1).reshape(KK, NB, FIN)
    xf = _dot(xgf, wdt).reshape(KK, NB, FOUT)
    return hf, xf


def _stats1_body(xg_ref, y1_ref, a_ref, w1t_ref, wxt_ref, b1_ref,
                 sh_ref, sx_ref):
    y1b = y1_ref[0]
    ab = a_ref[0]
    hf, xf = _edge_blocks(xg_ref, w1t_ref[...], b1_ref[...],
                          wxt_ref[...][FIN:])
    sh = jnp.zeros((1, FIN), jnp.float32)
    sh2 = jnp.zeros((1, FIN), jnp.float32)
    sx = jnp.zeros((1, FOUT), jnp.float32)
    sx2 = jnp.zeros((1, FOUT), jnp.float32)
    for k in range(KK):
        h = hf[k] - y1b
        sh = sh + jnp.sum(h, axis=0, keepdims=True)
        sh2 = sh2 + jnp.sum(h * h, axis=0, keepdims=True)
        xx = xf[k] + ab
        sx = sx + jnp.sum(xx, axis=0, keepdims=True)
        sx2 = sx2 + jnp.sum(xx * xx, axis=0, keepdims=True)
    ph = jnp.concatenate([sh, sh2, jnp.zeros((6, FIN), jnp.float32)], axis=0)
    px = jnp.concatenate([sx, sx2, jnp.zeros((6, FOUT), jnp.float32)], axis=0)
    first = (pl.program_id(0) == 0) & (pl.program_id(1) == 0)

    @pl.when(first)
    def _():
        sh_ref[...] = jnp.zeros_like(sh_ref)
        sx_ref[...] = jnp.zeros_like(sx_ref)

    sh_ref[...] += ph
    sx_ref[...] += px


def _stats1_call(xg4, y1t, at, w1t, wxt, b1r):
    nblk = NN // NB
    return pl.pallas_call(
        _stats1_body,
        grid=(BB, nblk),
        in_specs=[
            pl.BlockSpec((1, KK, NB, FIN), lambda b, i: (b, 0, i, 0)),
            pl.BlockSpec((1, NB, FIN), lambda b, i: (b, i, 0)),
            pl.BlockSpec((1, NB, FOUT), lambda b, i: (b, i, 0)),
            pl.BlockSpec((FIN, FIN), lambda b, i: (0, 0)),
            pl.BlockSpec((2 * FIN, FOUT), lambda b, i: (0, 0)),
            pl.BlockSpec((1, FIN), lambda b, i: (0, 0)),
        ],
        out_specs=[
            pl.BlockSpec((8, FIN), lambda b, i: (0, 0)),
            pl.BlockSpec((8, FOUT), lambda b, i: (0, 0)),
        ],
        out_shape=[
            jax.ShapeDtypeStruct((8, FIN), jnp.float32),
            jax.ShapeDtypeStruct((8, FOUT), jnp.float32),
        ],
    )(xg4, y1t, at, w1t, wxt, b1r)


def _bn_consts(sums, g, be, count):
    m = sums[0:1, :] / count
    v = sums[1:2, :] / count - m * m
    a = g / jnp.sqrt(v + EPS)
    return m, a, be


def _stats2_body(xg_ref, y1_ref, w1t_ref, b1_ref, shs_ref, g1_ref, be1_ref,
                 w2t_ref, b2_ref, shw_ref):
    y1b = y1_ref[0]
    m1, a1, be1 = _bn_consts(shs_ref[...], g1_ref[...], be1_ref[...], M_EDGES)
    xgf = xg_ref[0].reshape(KK * NB, FIN)
    hf = (_dot(xgf, w1t_ref[...]) + b1_ref[...]).reshape(KK, NB, FIN)
    u_list = [_lrelu(((hf[k] - y1b) - m1) * a1 + be1) for k in range(KK)]
    uf = jnp.concatenate(u_list, axis=0)
    hwf = _dot(uf, w2t_ref[...]) + b2_ref[...]
    s = jnp.sum(hwf, axis=0, keepdims=True)
    s2 = jnp.sum(hwf * hwf, axis=0, keepdims=True)
    p = jnp.concatenate([s, s2, jnp.zeros((6, FOUT), jnp.float32)], axis=0)
    first = (pl.program_id(0) == 0) & (pl.program_id(1) == 0)

    @pl.when(first)
    def _():
        shw_ref[...] = jnp.zeros_like(shw_ref)

    shw_ref[...] += p


def _stats2_call(xg4, y1t, w1t, b1r, shs, g1r, be1r, w2t, b2r):
    nblk = NN // NB
    return pl.pallas_call(
        _stats2_body,
        grid=(BB, nblk),
        in_specs=[
            pl.BlockSpec((1, KK, NB, FIN), lambda b, i: (b, 0, i, 0)),
            pl.BlockSpec((1, NB, FIN), lambda b, i: (b, i, 0)),
            pl.BlockSpec((FIN, FIN), lambda b, i: (0, 0)),
            pl.BlockSpec((1, FIN), lambda b, i: (0, 0)),
            pl.BlockSpec((8, FIN), lambda b, i: (0, 0)),
            pl.BlockSpec((1, FIN), lambda b, i: (0, 0)),
            pl.BlockSpec((1, FIN), lambda b, i: (0, 0)),
            pl.BlockSpec((FIN, FOUT), lambda b, i: (0, 0)),
            pl.BlockSpec((1, FOUT), lambda b, i: (0, 0)),
        ],
        out_specs=[pl.BlockSpec((8, FOUT), lambda b, i: (0, 0))],
        out_shape=[jax.ShapeDtypeStruct((8, FOUT), jnp.float32)],
    )(xg4, y1t, w1t, b1r, shs, g1r, be1r, w2t, b2r)[0]


def _main_body(xg_ref, y1_ref, a_ref, w1t_ref, wxt_ref, b1_ref,
               shs_ref, g1_ref, be1_ref, w2t_ref, b2_ref,
               shw_ref, g2_ref, be2_ref, sxs_ref, gx_ref, bex_ref,
               wot_ref, bo_ref, o_ref, ast_ref):
    y1b = y1_ref[0]
    ab = a_ref[0]
    m1, a1, be1 = _bn_consts(shs_ref[...], g1_ref[...], be1_ref[...], M_EDGES)
    m2, a2, be2 = _bn_consts(shw_ref[...], g2_ref[...], be2_ref[...], M_EDGES)
    m3, a3, bex = _bn_consts(sxs_ref[...], gx_ref[...], bex_ref[...], M_EDGES)
    hf, xf = _edge_blocks(xg_ref, w1t_ref[...], b1_ref[...],
                          wxt_ref[...][FIN:])
    u_list = [_lrelu(((hf[k] - y1b) - m1) * a1 + be1) for k in range(KK)]
    uf = jnp.concatenate(u_list, axis=0)
    hwf = _dot(uf, w2t_ref[...]) + b2_ref[...]
    hw4 = hwf.reshape(KK, NB, FOUT)
    z_list = []
    xxn_list = []
    for k in range(KK):
        z_list.append(_lrelu((hw4[k] - m2) * a2 + be2))
        xx = xf[k] + ab
        xxn_list.append(_lrelu((xx - m3) * a3 + bex))
    mx = z_list[0]
    for k in range(1, KK):
        mx = jnp.maximum(mx, z_list[k])
    e_list = []
    s = jnp.zeros((NB, FOUT), jnp.float32)
    for k in range(KK):
        e = jnp.exp(z_list[k] - mx)
        e_list.append(e)
        s = s + e
    rs = 1.0 / s
    acc = jnp.zeros((NB, FOUT), jnp.float32)
    for k in range(KK):
        acc = acc + _dot(xxn_list[k] * e_list[k] * rs, wot_ref[k])
    out = acc + bo_ref[...]
    o_ref[0] = out
    so = jnp.sum(out, axis=0, keepdims=True)
    so2 = jnp.sum(out * out, axis=0, keepdims=True)
    p = jnp.concatenate([so, so2, jnp.zeros((6, FOUT), jnp.float32)], axis=0)
    first = pl.program_id(1) == 0

    @pl.when(first)
    def _():
        ast_ref[...] = jnp.zeros_like(ast_ref)

    ast_ref[...] += p[None]


def _main_call(xg4, y1t, at, w1t, wxt, b1r, shs, g1r, be1r, w2t, b2r,
               shw, g2r, be2r, sxs, gxr, bexr, wot, bor):
    nblk = NN // NB
    cfull = lambda b, i: (0, 0)
    return pl.pallas_call(
        _main_body,
        grid=(BB, nblk),
        in_specs=[
            pl.BlockSpec((1, KK, NB, FIN), lambda b, i: (b, 0, i, 0)),
            pl.BlockSpec((1, NB, FIN), lambda b, i: (b, i, 0)),
            pl.BlockSpec((1, NB, FOUT), lambda b, i: (b, i, 0)),
            pl.BlockSpec((FIN, FIN), cfull),
            pl.BlockSpec((2 * FIN, FOUT), cfull),
            pl.BlockSpec((1, FIN), cfull),
            pl.BlockSpec((8, FIN), cfull),
            pl.BlockSpec((1, FIN), cfull),
            pl.BlockSpec((1, FIN), cfull),
            pl.BlockSpec((FIN, FOUT), cfull),
            pl.BlockSpec((1, FOUT), cfull),
            pl.BlockSpec((8, FOUT), cfull),
            pl.BlockSpec((1, FOUT), cfull),
            pl.BlockSpec((1, FOUT), cfull),
            pl.BlockSpec((8, FOUT), cfull),
            pl.BlockSpec((1, FOUT), cfull),
            pl.BlockSpec((1, FOUT), cfull),
            pl.BlockSpec((KK, FOUT, FOUT), lambda b, i: (0, 0, 0)),
            pl.BlockSpec((1, FOUT), cfull),
        ],
        out_specs=[
            pl.BlockSpec((1, NB, FOUT), lambda b, i: (b, i, 0)),
            pl.BlockSpec((1, 8, FOUT), lambda b, i: (b, 0, 0)),
        ],
        out_shape=[
            jax.ShapeDtypeStruct((BB, NN, FOUT), jnp.float32),
            jax.ShapeDtypeStruct((BB, 8, FOUT), jnp.float32),
        ],
    )(xg4, y1t, at, w1t, wxt, b1r, shs, g1r, be1r, w2t, b2r,
      shw, g2r, be2r, sxs, gxr, bexr, wot, bor)


def _ada_body(o_ref, st_ref, ast_ref, out_ref):
    stats = ast_ref[0]
    m = stats[0:1, :] / float(NN)
    v = stats[1:2, :] / float(NN) - m * m
    o = o_ref[0]
    sv = st_ref[0]
    gamma = sv[:, :FOUT]
    beta = sv[:, FOUT:]
    y = gamma * (o - m) / jnp.sqrt(v + EPS) + beta
    out_ref[0] = _lrelu(y)


def _ada_call(o, st, ast):
    nblk = NN // NB
    return pl.pallas_call(
        _ada_body,
        grid=(BB, nblk),
        in_specs=[
            pl.BlockSpec((1, NB, FOUT), lambda b, i: (b, i, 0)),
            pl.BlockSpec((1, NB, 2 * FOUT), lambda b, i: (b, i, 0)),
            pl.BlockSpec((1, 8, FOUT), lambda b, i: (b, 0, 0)),
        ],
        out_specs=pl.BlockSpec((1, NB, FOUT), lambda b, i: (b, i, 0)),
        out_shape=jax.ShapeDtypeStruct((BB, NN, FOUT), jnp.float32),
    )(o, st, ast)


def kernel(x, style, W1, b1, g1, be1, W2, b2, g2, be2, Wx, bx, gx, bex,
           Wo, bo, Ws, bs):
    xt = jnp.transpose(x, (0, 2, 1))
    stt = jnp.transpose(style, (0, 2, 1))
    w1t = W1.T
    wxt = Wx.T
    w2t = W2.T
    wst = Ws.T
    wot = jnp.transpose(Wo, (2, 1, 0))
    b1r = b1.reshape(1, FIN)
    g1r = g1.reshape(1, FIN)
    be1r = be1.reshape(1, FIN)
    b2r = b2.reshape(1, FOUT)
    g2r = g2.reshape(1, FOUT)
    be2r = be2.reshape(1, FOUT)
    bxr = bx.reshape(1, FOUT)
    gxr = gx.reshape(1, FOUT)
    bexr = bex.reshape(1, FOUT)
    bor = bo.reshape(1, FOUT)
    bsr = bs.reshape(1, 2 * FOUT)

    idx, y1t, at, st = _prep_call(xt, x, stt, w1t, wxt, wst, bxr, bsr)

    off = (jnp.arange(BB, dtype=jnp.int32) * NN).reshape(BB, 1, 1)
    flat = (jnp.transpose(idx, (0, 2, 1)) + off).reshape(NW, NCH_W, CHUNK)
    xg = _gather_call(xt.reshape(BB * NN, FIN), flat)
    xg4 = xg.reshape(BB, KK, NN, FIN)

    shs, sxs = _stats1_call(xg4, y1t, at, w1t, wxt, b1r)
    shw = _stats2_call(xg4, y1t, w1t, b1r, shs, g1r, be1r, w2t, b2r)
    o, ast = _main_call(xg4, y1t, at, w1t, wxt, b1r, shs, g1r, be1r, w2t,
                        b2r, shw, g2r, be2r, sxs, gxr, bexr, wot, bor)
    outf = _ada_call(o, st, ast)
    return jnp.transpose(outf, (0, 2, 1))
